# baseline (speedup 1.0000x reference)
import functools

import jax
import jax.numpy as jnp
from jax import lax
from jax.experimental import pallas as pl
from jax.experimental.pallas import tpu as pltpu
from jax.experimental.pallas import tpu_sc as plsc

T = 2048
E = 8
H = 768
I = 1024
K = 2
P = T * K
BLOCK = 256
NBLK = P // BLOCK + E
ROWS = NBLK * BLOCK
LOG2_BLOCK = 8

NTILES = 16
TPT = T // NTILES
ZCH = ROWS // NTILES
NW = 32
RPW = ROWS // NW
GCH = 64
TPW = T // NW

_LANE = None


def _splat(ref, e):
    return plsc.load_gather(ref, [jnp.full((16,), e, jnp.int32)])


def _routing_body(gt_hbm, tok_hbm, wsort_hbm, pos_hbm, incl_hbm, cnts_hbm,
                  g_v, e0_v, e1_v, w0_v, w1_v, p0_v, p1_v, tok_v,
                  cnt_v, allcnt_v, base_v, incl_v, zi_v, zf_v, sem):
    c = lax.axis_index("c")

    @pl.when(c == 0)
    def _():
        w = lax.axis_index("s")
        base = w * TPT
        lane = lax.iota(jnp.int32, 16)
        wv = jnp.broadcast_to(w, (16,)).astype(jnp.int32)
        ones16 = jnp.ones((16,), jnp.int32)

        pltpu.sync_copy(gt_hbm, g_v)
        cnt_v[...] = jnp.zeros((16,), jnp.int32)

        def pass_a(j, carry):
            g = [g_v[e, pl.ds(base + j * 16, 16)] for e in range(E)]
            m1 = g[0]
            i1 = jnp.zeros((16,), jnp.int32)
            for e in range(1, E):
                gt = g[e] > m1
                m1 = jnp.where(gt, g[e], m1)
                i1 = jnp.where(gt, e, i1)
            m2 = jnp.full((16,), -jnp.inf, jnp.float32)
            i2 = jnp.zeros((16,), jnp.int32)
            for e in range(E):
                gt = (i1 != e) & (g[e] > m2)
                m2 = jnp.where(gt, g[e], m2)
                i2 = jnp.where(gt, e, i2)
            wt1 = 1.0 / (1.0 + jnp.exp(m2 - m1))
            sl = pl.ds(j * 16, 16)
            e0_v[sl] = i1
            e1_v[sl] = i2
            w0_v[sl] = wt1
            w1_v[sl] = 1.0 - wt1
            tok_v[sl] = base + j * 16 + lane
            plsc.addupdate_scatter(cnt_v, [i1], ones16)
            plsc.addupdate_scatter(cnt_v, [i2], ones16)
            return carry

        lax.fori_loop(0, TPT // 16, pass_a, 0)

        pltpu.sync_copy(cnt_v, cnts_hbm.at[pl.ds(w * 16, 16)])

        def zinit(j, carry):
            sl = pl.ds(j * 16, 16)
            zi_v[sl] = jnp.zeros((16,), jnp.int32)
            zf_v[sl] = jnp.zeros((16,), jnp.float32)
            return carry

        lax.fori_loop(0, ZCH // 16, zinit, 0)
        pltpu.sync_copy(zi_v, tok_hbm.at[pl.ds(w * ZCH, ZCH)])
        pltpu.sync_copy(zf_v, wsort_hbm.at[pl.ds(w * ZCH, ZCH)])

    plsc.subcore_barrier()

    @pl.when(c == 0)
    def _():
        w = lax.axis_index("s")
        base = w * TPT
        lane = lax.iota(jnp.int32, 16)
        wv = jnp.broadcast_to(w, (16,)).astype(jnp.int32)
        ones16 = jnp.ones((16,), jnp.int32)

        pltpu.sync_copy(cnts_hbm, allcnt_v)
        totals = jnp.zeros((16,), jnp.int32)
        myprefix = jnp.zeros((16,), jnp.int32)
        for r in range(NTILES):
            row = allcnt_v[pl.ds(r * 16, 16)]
            totals = totals + row
            rv = jnp.full((16,), r, jnp.int32)
            myprefix = myprefix + jnp.where(rv < wv, row, 0)
        padded = ((totals + (BLOCK - 1)) >> LOG2_BLOCK) << LOG2_BLOCK
        incl = plsc.cumsum(padded)
        incl_v[...] = incl
        base_v[...] = (incl - padded) + myprefix

        def mk_pass_b(ev_ref, pv_ref):
            def pass_b(j, carry):
                sl = pl.ds(j * 16, 16)
                ev = ev_ref[sl]
                rank = jnp.zeros((16,), jnp.int32)
                for e in range(E):
                    m = ev == e
                    cs = plsc.cumsum(m.astype(jnp.int32))
                    rank = jnp.where(m, cs - 1, rank)
                pv_ref[sl] = plsc.load_gather(base_v, [ev]) + rank
                plsc.addupdate_scatter(base_v, [ev], ones16)
                return carry
            return pass_b

        lax.fori_loop(0, TPT // 16, mk_pass_b(e0_v, p0_v), 0)
        lax.fori_loop(0, TPT // 16, mk_pass_b(e1_v, p1_v), 0)

        pltpu.async_copy(tok_v, tok_hbm.at[p0_v], sem).wait()
        pltpu.async_copy(tok_v, tok_hbm.at[p1_v], sem).wait()
        pltpu.async_copy(w0_v, wsort_hbm.at[p0_v], sem).wait()
        pltpu.async_copy(w1_v, wsort_hbm.at[p1_v], sem).wait()
        pltpu.sync_copy(p0_v, pos_hbm.at[0, pl.ds(base, TPT)])
        pltpu.sync_copy(p1_v, pos_hbm.at[1, pl.ds(base, TPT)])

        @pl.when(w == 0)
        def _():
            pltpu.sync_copy(incl_v, incl_hbm)


def _routing(gt):
    mesh = plsc.VectorSubcoreMesh(core_axis_name="c", subcore_axis_name="s")
    f = pl.kernel(
        _routing_body,
        out_type=(
            jax.ShapeDtypeStruct((ROWS,), jnp.int32),
            jax.ShapeDtypeStruct((ROWS,), jnp.float32),
            jax.ShapeDtypeStruct((K, T), jnp.int32),
            jax.ShapeDtypeStruct((16,), jnp.int32),
            jax.ShapeDtypeStruct((NTILES * 16,), jnp.int32),
        ),
        mesh=mesh,
        scratch_types=[
            pltpu.VMEM((E, T), jnp.float32),
            pltpu.VMEM((TPT,), jnp.int32),
            pltpu.VMEM((TPT,), jnp.int32),
            pltpu.VMEM((TPT,), jnp.float32),
            pltpu.VMEM((TPT,), jnp.float32),
            pltpu.VMEM((TPT,), jnp.int32),
            pltpu.VMEM((TPT,), jnp.int32),
            pltpu.VMEM((TPT,), jnp.int32),
            pltpu.VMEM((16,), jnp.int32),
            pltpu.VMEM((NTILES * 16,), jnp.int32),
            pltpu.VMEM((16,), jnp.int32),
            pltpu.VMEM((16,), jnp.int32),
            pltpu.VMEM((ZCH,), jnp.int32),
            pltpu.VMEM((ZCH,), jnp.float32),
            pltpu.SemaphoreType.DMA,
        ],
        compiler_params=pltpu.CompilerParams(needs_layout_passes=False),
    )
    return f(gt)[:4]


def _gather_body(hs_hbm, tok_hbm, xs_hbm, idx_v, rows_v, sem):
    wid = lax.axis_index("s") * 2 + lax.axis_index("c")
    for ch in range(RPW // GCH):
        b = wid * RPW + ch * GCH
        pltpu.sync_copy(tok_hbm.at[pl.ds(b, GCH)], idx_v)
        pltpu.async_copy(hs_hbm.at[idx_v], rows_v, sem).wait()
        pltpu.sync_copy(rows_v, xs_hbm.at[pl.ds(b, GCH)])


def _gather(hs, tok_sorted):
    mesh = plsc.VectorSubcoreMesh(core_axis_name="c", subcore_axis_name="s")
    f = pl.kernel(
        _gather_body,
        out_type=jax.ShapeDtypeStruct((ROWS, H), jnp.float32),
        mesh=mesh,
        scratch_types=[
            pltpu.VMEM((GCH,), jnp.int32),
            pltpu.VMEM((GCH, H), jnp.float32),
            pltpu.SemaphoreType.DMA,
        ],
        compiler_params=pltpu.CompilerParams(needs_layout_passes=False),
    )
    return f(hs, tok_sorted)


def _gmm_body(incl_ref, x_ref, w1_ref, w2_ref, ws_ref, o_ref):
    b = pl.program_id(0)

    @pl.when(b < (incl_ref[E - 1] >> LOG2_BLOCK))
    def _():
        x = x_ref[...]
        h = lax.dot_general(x, w1_ref[0], (((1,), (1,)), ((), ())),
                            preferred_element_type=jnp.float32)
        x1 = h[:, :I]
        x2 = h[:, I:]
        act = x1 * lax.logistic(x1) * x2
        o = lax.dot_general(act, w2_ref[0], (((1,), (1,)), ((), ())),
                            preferred_element_type=jnp.float32)
        o_ref[...] = o * ws_ref[...]


def _clamp_blk(b, r):
    tb = r[E - 1] >> LOG2_BLOCK
    return jnp.where(b < tb, b, tb - 1)


def _exp_of(b, r):
    bc = _clamp_blk(b, r) * BLOCK
    e = jnp.int32(0)
    for i in range(E):
        e = e + (bc >= r[i]).astype(jnp.int32)
    return e


def _gmm(x_sorted, w1, w2, w_sorted, incl16):
    grid_spec = pltpu.PrefetchScalarGridSpec(
        num_scalar_prefetch=1,
        grid=(NBLK,),
        in_specs=[
            pl.BlockSpec((BLOCK, H), lambda b, r: (_clamp_blk(b, r), 0)),
            pl.BlockSpec((1, 2 * I, H), lambda b, r: (_exp_of(b, r), 0, 0)),
            pl.BlockSpec((1, H, I), lambda b, r: (_exp_of(b, r), 0, 0)),
            pl.BlockSpec((BLOCK, 1), lambda b, r: (_clamp_blk(b, r), 0)),
        ],
        out_specs=pl.BlockSpec((BLOCK, H), lambda b, r: (b, 0)),
    )
    return pl.pallas_call(
        _gmm_body,
        grid_spec=grid_spec,
        out_shape=jax.ShapeDtypeStruct((ROWS, H), jnp.float32),
    )(incl16, x_sorted, w1, w2, w_sorted.reshape(ROWS, 1))


def _combine_body(os_hbm, pos_hbm, out_hbm, idx_v, a_v, b_v, sem):
    wid = lax.axis_index("s") * 2 + lax.axis_index("c")
    t0 = wid * TPW
    pltpu.sync_copy(pos_hbm.at[0, pl.ds(t0, TPW)], idx_v)
    pltpu.async_copy(os_hbm.at[idx_v], a_v, sem).wait()
    pltpu.sync_copy(pos_hbm.at[1, pl.ds(t0, TPW)], idx_v)
    pltpu.async_copy(os_hbm.at[idx_v], b_v, sem).wait()

    def body(i, carry):
        for ch in range(H // 16):
            sl = pl.ds(ch * 16, 16)
            a_v[i, sl] = a_v[i, sl] + b_v[i, sl]
        return carry

    lax.fori_loop(0, TPW, body, 0)
    pltpu.sync_copy(a_v, out_hbm.at[pl.ds(t0, TPW)])


def _combine(out_sorted, pos):
    mesh = plsc.VectorSubcoreMesh(core_axis_name="c", subcore_axis_name="s")
    f = pl.kernel(
        _combine_body,
        out_type=jax.ShapeDtypeStruct((T, H), jnp.float32),
        mesh=mesh,
        scratch_types=[
            pltpu.VMEM((TPW,), jnp.int32),
            pltpu.VMEM((TPW, H), jnp.float32),
            pltpu.VMEM((TPW, H), jnp.float32),
            pltpu.SemaphoreType.DMA,
        ],
        compiler_params=pltpu.CompilerParams(needs_layout_passes=False),
    )
    return f(out_sorted, pos)


def kernel(hidden_states, w1, w2, gating_output):
    orig_shape = hidden_states.shape
    hs = hidden_states.reshape(T, H)
    gt = gating_output.reshape(T, E).T
    tok_sorted, w_sorted, pos, incl16 = _routing(gt)
    x_sorted = _gather(hs, tok_sorted)
    out_sorted = _gmm(x_sorted, w1, w2, w_sorted, incl16)
    out = _combine(out_sorted, pos)
    return out.reshape(orig_shape)


# speedup vs baseline: 1.0176x; 1.0176x over previous
import functools

import jax
import jax.numpy as jnp
from jax import lax
from jax.experimental import pallas as pl
from jax.experimental.pallas import tpu as pltpu
from jax.experimental.pallas import tpu_sc as plsc

T = 2048
E = 8
H = 768
I = 1024
K = 2
P = T * K
BLOCK = 256
NBLK = P // BLOCK + E
ROWS = NBLK * BLOCK
LOG2_BLOCK = 8

NTILES = 16
TPT = T // NTILES
ZCH = ROWS // NTILES
NW = 32
RPW = ROWS // NW
GCH = 64
TPW = T // NW

_LANE = None


def _splat(ref, e):
    return plsc.load_gather(ref, [jnp.full((16,), e, jnp.int32)])


def _routing_body(gt_hbm, tok_hbm, wsort_hbm, pos_hbm, incl_hbm, cnts_hbm,
                  g_v, e0_v, e1_v, w0_v, w1_v, p0_v, p1_v, tok_v,
                  cnt_v, allcnt_v, base_v, incl_v, zi_v, zf_v,
                  sem, sem2, sem3, sem4, sem5, sem6):
    c = lax.axis_index("c")

    @pl.when(c == 0)
    def _():
        w = lax.axis_index("s")
        base = w * TPT
        lane = lax.iota(jnp.int32, 16)
        wv = jnp.broadcast_to(w, (16,)).astype(jnp.int32)
        ones16 = jnp.ones((16,), jnp.int32)

        pltpu.sync_copy(gt_hbm, g_v)
        cnt_v[...] = jnp.zeros((16,), jnp.int32)

        def pass_a(j, carry):
            g = [g_v[e, pl.ds(base + j * 16, 16)] for e in range(E)]
            m1 = g[0]
            i1 = jnp.zeros((16,), jnp.int32)
            for e in range(1, E):
                gt = g[e] > m1
                m1 = jnp.where(gt, g[e], m1)
                i1 = jnp.where(gt, e, i1)
            m2 = jnp.full((16,), -jnp.inf, jnp.float32)
            i2 = jnp.zeros((16,), jnp.int32)
            for e in range(E):
                gt = (i1 != e) & (g[e] > m2)
                m2 = jnp.where(gt, g[e], m2)
                i2 = jnp.where(gt, e, i2)
            wt1 = 1.0 / (1.0 + jnp.exp(m2 - m1))
            sl = pl.ds(j * 16, 16)
            e0_v[sl] = i1
            e1_v[sl] = i2
            w0_v[sl] = wt1
            w1_v[sl] = 1.0 - wt1
            tok_v[sl] = base + j * 16 + lane
            plsc.addupdate_scatter(cnt_v, [i1], ones16)
            plsc.addupdate_scatter(cnt_v, [i2], ones16)
            return carry

        lax.fori_loop(0, TPT // 16, pass_a, 0)

        pltpu.sync_copy(cnt_v, cnts_hbm.at[pl.ds(w * 16, 16)])

        def zinit(j, carry):
            sl = pl.ds(j * 16, 16)
            zi_v[sl] = jnp.zeros((16,), jnp.int32)
            zf_v[sl] = jnp.zeros((16,), jnp.float32)
            return carry

        lax.fori_loop(0, ZCH // 16, zinit, 0)
        z1 = pltpu.async_copy(zi_v, tok_hbm.at[pl.ds(w * ZCH, ZCH)], sem)
        z2 = pltpu.async_copy(zf_v, wsort_hbm.at[pl.ds(w * ZCH, ZCH)], sem2)
        z1.wait()
        z2.wait()

    plsc.subcore_barrier()

    @pl.when(c == 0)
    def _():
        w = lax.axis_index("s")
        base = w * TPT
        lane = lax.iota(jnp.int32, 16)
        wv = jnp.broadcast_to(w, (16,)).astype(jnp.int32)
        ones16 = jnp.ones((16,), jnp.int32)

        pltpu.sync_copy(cnts_hbm, allcnt_v)
        totals = jnp.zeros((16,), jnp.int32)
        myprefix = jnp.zeros((16,), jnp.int32)
        for r in range(NTILES):
            row = allcnt_v[pl.ds(r * 16, 16)]
            totals = totals + row
            rv = jnp.full((16,), r, jnp.int32)
            myprefix = myprefix + jnp.where(rv < wv, row, 0)
        padded = ((totals + (BLOCK - 1)) >> LOG2_BLOCK) << LOG2_BLOCK
        incl = plsc.cumsum(padded)
        incl_v[...] = incl
        base_v[...] = (incl - padded) + myprefix

        def mk_pass_b(ev_ref, pv_ref):
            def pass_b(j, carry):
                sl = pl.ds(j * 16, 16)
                ev = ev_ref[sl]
                rank = jnp.zeros((16,), jnp.int32)
                for e in range(E):
                    m = ev == e
                    cs = plsc.cumsum(m.astype(jnp.int32))
                    rank = jnp.where(m, cs - 1, rank)
                pv_ref[sl] = plsc.load_gather(base_v, [ev]) + rank
                plsc.addupdate_scatter(base_v, [ev], ones16)
                return carry
            return pass_b

        lax.fori_loop(0, TPT // 16, mk_pass_b(e0_v, p0_v), 0)
        lax.fori_loop(0, TPT // 16, mk_pass_b(e1_v, p1_v), 0)

        c1 = pltpu.async_copy(tok_v, tok_hbm.at[p0_v], sem)
        c2 = pltpu.async_copy(tok_v, tok_hbm.at[p1_v], sem2)
        c3 = pltpu.async_copy(w0_v, wsort_hbm.at[p0_v], sem3)
        c4 = pltpu.async_copy(w1_v, wsort_hbm.at[p1_v], sem4)
        c5 = pltpu.async_copy(p0_v, pos_hbm.at[0, pl.ds(base, TPT)], sem5)
        c6 = pltpu.async_copy(p1_v, pos_hbm.at[1, pl.ds(base, TPT)], sem6)
        c1.wait()
        c2.wait()
        c3.wait()
        c4.wait()
        c5.wait()
        c6.wait()

        @pl.when(w == 0)
        def _():
            pltpu.sync_copy(incl_v, incl_hbm)


def _routing(gt):
    mesh = plsc.VectorSubcoreMesh(core_axis_name="c", subcore_axis_name="s")
    f = pl.kernel(
        _routing_body,
        out_type=(
            jax.ShapeDtypeStruct((ROWS,), jnp.int32),
            jax.ShapeDtypeStruct((ROWS,), jnp.float32),
            jax.ShapeDtypeStruct((K, T), jnp.int32),
            jax.ShapeDtypeStruct((16,), jnp.int32),
            jax.ShapeDtypeStruct((NTILES * 16,), jnp.int32),
        ),
        mesh=mesh,
        scratch_types=[
            pltpu.VMEM((E, T), jnp.float32),
            pltpu.VMEM((TPT,), jnp.int32),
            pltpu.VMEM((TPT,), jnp.int32),
            pltpu.VMEM((TPT,), jnp.float32),
            pltpu.VMEM((TPT,), jnp.float32),
            pltpu.VMEM((TPT,), jnp.int32),
            pltpu.VMEM((TPT,), jnp.int32),
            pltpu.VMEM((TPT,), jnp.int32),
            pltpu.VMEM((16,), jnp.int32),
            pltpu.VMEM((NTILES * 16,), jnp.int32),
            pltpu.VMEM((16,), jnp.int32),
            pltpu.VMEM((16,), jnp.int32),
            pltpu.VMEM((ZCH,), jnp.int32),
            pltpu.VMEM((ZCH,), jnp.float32),
            pltpu.SemaphoreType.DMA,
            pltpu.SemaphoreType.DMA,
            pltpu.SemaphoreType.DMA,
            pltpu.SemaphoreType.DMA,
            pltpu.SemaphoreType.DMA,
            pltpu.SemaphoreType.DMA,
        ],
        compiler_params=pltpu.CompilerParams(needs_layout_passes=False),
    )
    return f(gt)[:4]


def _gather_body(hs_hbm, tok_hbm, xs_hbm, idx_v, rows_a, rows_b,
                 sem_a, sem_b, sem_wa, sem_wb):
    wid = lax.axis_index("s") * 2 + lax.axis_index("c")
    base = wid * RPW
    pltpu.sync_copy(tok_hbm.at[pl.ds(base, RPW)], idx_v)
    g0 = pltpu.async_copy(hs_hbm.at[idx_v.at[pl.ds(0, GCH)]], rows_a, sem_a)
    g1 = pltpu.async_copy(hs_hbm.at[idx_v.at[pl.ds(GCH, GCH)]], rows_b, sem_b)
    g0.wait()
    w0 = pltpu.async_copy(rows_a, xs_hbm.at[pl.ds(base, GCH)], sem_wa)
    g1.wait()
    w1 = pltpu.async_copy(rows_b, xs_hbm.at[pl.ds(base + GCH, GCH)], sem_wb)
    w0.wait()
    g2 = pltpu.async_copy(hs_hbm.at[idx_v.at[pl.ds(2 * GCH, GCH)]], rows_a, sem_a)
    g2.wait()
    w2 = pltpu.async_copy(rows_a, xs_hbm.at[pl.ds(base + 2 * GCH, GCH)], sem_wa)
    w1.wait()
    w2.wait()


def _gather(hs, tok_sorted):
    mesh = plsc.VectorSubcoreMesh(core_axis_name="c", subcore_axis_name="s")
    f = pl.kernel(
        _gather_body,
        out_type=jax.ShapeDtypeStruct((ROWS, H), jnp.float32),
        mesh=mesh,
        scratch_types=[
            pltpu.VMEM((RPW,), jnp.int32),
            pltpu.VMEM((GCH, H), jnp.float32),
            pltpu.VMEM((GCH, H), jnp.float32),
            pltpu.SemaphoreType.DMA,
            pltpu.SemaphoreType.DMA,
            pltpu.SemaphoreType.DMA,
            pltpu.SemaphoreType.DMA,
        ],
        compiler_params=pltpu.CompilerParams(needs_layout_passes=False),
    )
    return f(hs, tok_sorted)


def _gmm_body(incl_ref, x_ref, w1_ref, w2_ref, ws_ref, o_ref, w1b_v, w2b_v):
    b = pl.program_id(0)
    tb = incl_ref[E - 1] >> LOG2_BLOCK

    @pl.when(b < tb)
    def _():
        eb = _exp_of(b, incl_ref)
        ebp = _exp_of(b - 1, incl_ref)

        @pl.when((b == 0) | (eb != ebp))
        def _():
            w1b_v[...] = w1_ref[0].astype(jnp.bfloat16)
            w2b_v[...] = w2_ref[0].astype(jnp.bfloat16)

        xb = x_ref[...].astype(jnp.bfloat16)
        h = lax.dot_general(xb, w1b_v[...], (((1,), (1,)), ((), ())),
                            preferred_element_type=jnp.float32)
        x1 = h[:, :I]
        x2 = h[:, I:]
        act = (x1 * lax.logistic(x1) * x2).astype(jnp.bfloat16)
        o = lax.dot_general(act, w2b_v[...], (((1,), (1,)), ((), ())),
                            preferred_element_type=jnp.float32)
        o_ref[...] = o * ws_ref[...]


def _clamp_blk(b, r):
    tb = r[E - 1] >> LOG2_BLOCK
    return jnp.where(b < tb, b, tb - 1)


def _exp_of(b, r):
    bc = _clamp_blk(b, r) * BLOCK
    e = jnp.int32(0)
    for i in range(E):
        e = e + (bc >= r[i]).astype(jnp.int32)
    return e


def _gmm(x_sorted, w1, w2, w_sorted, incl16):
    grid_spec = pltpu.PrefetchScalarGridSpec(
        num_scalar_prefetch=1,
        grid=(NBLK,),
        in_specs=[
            pl.BlockSpec((BLOCK, H), lambda b, r: (_clamp_blk(b, r), 0)),
            pl.BlockSpec((1, 2 * I, H), lambda b, r: (_exp_of(b, r), 0, 0)),
            pl.BlockSpec((1, H, I), lambda b, r: (_exp_of(b, r), 0, 0)),
            pl.BlockSpec((BLOCK, 1), lambda b, r: (_clamp_blk(b, r), 0)),
        ],
        out_specs=pl.BlockSpec((BLOCK, H), lambda b, r: (b, 0)),
        scratch_shapes=[
            pltpu.VMEM((2 * I, H), jnp.bfloat16),
            pltpu.VMEM((H, I), jnp.bfloat16),
        ],
    )
    return pl.pallas_call(
        _gmm_body,
        grid_spec=grid_spec,
        out_shape=jax.ShapeDtypeStruct((ROWS, H), jnp.float32),
    )(incl16, x_sorted, w1, w2, w_sorted.reshape(ROWS, 1))


def _combine_body(os_hbm, pos_hbm, out_hbm, idx_v, a_v, b_v, sem):
    wid = lax.axis_index("s") * 2 + lax.axis_index("c")
    t0 = wid * TPW
    pltpu.sync_copy(pos_hbm.at[0, pl.ds(t0, TPW)], idx_v)
    pltpu.async_copy(os_hbm.at[idx_v], a_v, sem).wait()
    pltpu.sync_copy(pos_hbm.at[1, pl.ds(t0, TPW)], idx_v)
    pltpu.async_copy(os_hbm.at[idx_v], b_v, sem).wait()

    def body(i, carry):
        for ch in range(H // 16):
            sl = pl.ds(ch * 16, 16)
            a_v[i, sl] = a_v[i, sl] + b_v[i, sl]
        return carry

    lax.fori_loop(0, TPW, body, 0)
    pltpu.sync_copy(a_v, out_hbm.at[pl.ds(t0, TPW)])


def _combine(out_sorted, pos):
    mesh = plsc.VectorSubcoreMesh(core_axis_name="c", subcore_axis_name="s")
    f = pl.kernel(
        _combine_body,
        out_type=jax.ShapeDtypeStruct((T, H), jnp.float32),
        mesh=mesh,
        scratch_types=[
            pltpu.VMEM((TPW,), jnp.int32),
            pltpu.VMEM((TPW, H), jnp.float32),
            pltpu.VMEM((TPW, H), jnp.float32),
            pltpu.SemaphoreType.DMA,
        ],
        compiler_params=pltpu.CompilerParams(needs_layout_passes=False),
    )
    return f(out_sorted, pos)


def kernel(hidden_states, w1, w2, gating_output):
    orig_shape = hidden_states.shape
    hs = hidden_states.reshape(T, H)
    gt = gating_output.reshape(T, E).T
    tok_sorted, w_sorted, pos, incl16 = _routing(gt)
    x_sorted = _gather(hs, tok_sorted)
    out_sorted = _gmm(x_sorted, w1, w2, w_sorted, incl16)
    out = _combine(out_sorted, pos)
    return out.reshape(orig_shape)


# speedup vs baseline: 1.6281x; 1.5999x over previous
import functools

import jax
import jax.numpy as jnp
from jax import lax
from jax.experimental import pallas as pl
from jax.experimental.pallas import tpu as pltpu
from jax.experimental.pallas import tpu_sc as plsc

T = 2048
E = 8
H = 768
I = 1024
K = 2
P = T * K
BLOCK = 256
NBLK = P // BLOCK + E
ROWS = NBLK * BLOCK
LOG2_BLOCK = 8

NTILES = 16
TPT = T // NTILES
ZCH = ROWS // NTILES
NW = 32
RPW = ROWS // NW
GCH = 64
TPW = T // NW

_LANE = None


def _splat(ref, e):
    return plsc.load_gather(ref, [jnp.full((16,), e, jnp.int32)])


def _routing_body(gt_hbm, tok_hbm, wsort_hbm, pos_hbm, incl_hbm, cnts_hbm,
                  g_v, e0_v, e1_v, w0_v, w1_v, p0_v, p1_v, tok_v,
                  cnt_v, allcnt_v, base_v, incl_v, zi_v, zf_v,
                  sem, sem2, sem3, sem4, sem5, sem6):
    c = lax.axis_index("c")

    @pl.when(c == 0)
    def _():
        w = lax.axis_index("s")
        base = w * TPT
        lane = lax.iota(jnp.int32, 16)
        wv = jnp.broadcast_to(w, (16,)).astype(jnp.int32)
        ones16 = jnp.ones((16,), jnp.int32)

        pltpu.sync_copy(gt_hbm, g_v)
        cnt_v[...] = jnp.zeros((16,), jnp.int32)

        def pass_a(j, carry):
            g = [g_v[e, pl.ds(base + j * 16, 16)] for e in range(E)]
            m1 = g[0]
            i1 = jnp.zeros((16,), jnp.int32)
            for e in range(1, E):
                gt = g[e] > m1
                m1 = jnp.where(gt, g[e], m1)
                i1 = jnp.where(gt, e, i1)
            m2 = jnp.full((16,), -jnp.inf, jnp.float32)
            i2 = jnp.zeros((16,), jnp.int32)
            for e in range(E):
                gt = (i1 != e) & (g[e] > m2)
                m2 = jnp.where(gt, g[e], m2)
                i2 = jnp.where(gt, e, i2)
            wt1 = 1.0 / (1.0 + jnp.exp(m2 - m1))
            sl = pl.ds(j * 16, 16)
            e0_v[sl] = i1
            e1_v[sl] = i2
            w0_v[sl] = wt1
            w1_v[sl] = 1.0 - wt1
            tok_v[sl] = base + j * 16 + lane
            plsc.addupdate_scatter(cnt_v, [i1], ones16)
            plsc.addupdate_scatter(cnt_v, [i2], ones16)
            return carry

        lax.fori_loop(0, TPT // 16, pass_a, 0)

        cpub = pltpu.async_copy(cnt_v, cnts_hbm.at[pl.ds(w * 16, 16)], sem5)

        def zinit(j, carry):
            sl = pl.ds(j * 16, 16)
            zi_v[sl] = (w * ZCH + j * 16 + lane) & (T - 1)
            zf_v[sl] = jnp.zeros((16,), jnp.float32)
            return carry

        lax.fori_loop(0, ZCH // 16, zinit, 0)
        z1 = pltpu.async_copy(zi_v, tok_hbm.at[pl.ds(w * ZCH, ZCH)], sem)
        z2 = pltpu.async_copy(zf_v, wsort_hbm.at[pl.ds(w * ZCH, ZCH)], sem2)
        z1.wait()
        z2.wait()
        cpub.wait()

    plsc.subcore_barrier()

    @pl.when(c == 0)
    def _():
        w = lax.axis_index("s")
        base = w * TPT
        lane = lax.iota(jnp.int32, 16)
        wv = jnp.broadcast_to(w, (16,)).astype(jnp.int32)
        ones16 = jnp.ones((16,), jnp.int32)

        pltpu.sync_copy(cnts_hbm, allcnt_v)
        totals = jnp.zeros((16,), jnp.int32)
        myprefix = jnp.zeros((16,), jnp.int32)
        for r in range(NTILES):
            row = allcnt_v[pl.ds(r * 16, 16)]
            totals = totals + row
            rv = jnp.full((16,), r, jnp.int32)
            myprefix = myprefix + jnp.where(rv < wv, row, 0)
        padded = ((totals + (BLOCK - 1)) >> LOG2_BLOCK) << LOG2_BLOCK
        incl = plsc.cumsum(padded)
        incl_v[...] = incl
        base_v[...] = (incl - padded) + myprefix

        def mk_pass_b(ev_ref, pv_ref):
            def pass_b(j, carry):
                sl = pl.ds(j * 16, 16)
                ev = ev_ref[sl]
                rank = jnp.zeros((16,), jnp.int32)
                for e in range(E):
                    m = ev == e
                    cs = plsc.cumsum(m.astype(jnp.int32))
                    rank = jnp.where(m, cs - 1, rank)
                pv_ref[sl] = plsc.load_gather(base_v, [ev]) + rank
                plsc.addupdate_scatter(base_v, [ev], ones16)
                return carry
            return pass_b

        lax.fori_loop(0, TPT // 16, mk_pass_b(e0_v, p0_v), 0)
        lax.fori_loop(0, TPT // 16, mk_pass_b(e1_v, p1_v), 0)

        c1 = pltpu.async_copy(tok_v, tok_hbm.at[p0_v], sem)
        c2 = pltpu.async_copy(tok_v, tok_hbm.at[p1_v], sem2)
        c3 = pltpu.async_copy(w0_v, wsort_hbm.at[p0_v], sem3)
        c4 = pltpu.async_copy(w1_v, wsort_hbm.at[p1_v], sem4)
        c5 = pltpu.async_copy(p0_v, pos_hbm.at[0, pl.ds(base, TPT)], sem5)
        c6 = pltpu.async_copy(p1_v, pos_hbm.at[1, pl.ds(base, TPT)], sem6)
        c1.wait()
        c2.wait()
        c3.wait()
        c4.wait()
        c5.wait()
        c6.wait()

        @pl.when(w == 0)
        def _():
            pltpu.sync_copy(incl_v, incl_hbm)


def _routing(gt):
    mesh = plsc.VectorSubcoreMesh(core_axis_name="c", subcore_axis_name="s")
    f = pl.kernel(
        _routing_body,
        out_type=(
            jax.ShapeDtypeStruct((ROWS,), jnp.int32),
            jax.ShapeDtypeStruct((ROWS,), jnp.float32),
            jax.ShapeDtypeStruct((K, T), jnp.int32),
            jax.ShapeDtypeStruct((16,), jnp.int32),
            jax.ShapeDtypeStruct((NTILES * 16,), jnp.int32),
        ),
        mesh=mesh,
        scratch_types=[
            pltpu.VMEM((E, T), jnp.float32),
            pltpu.VMEM((TPT,), jnp.int32),
            pltpu.VMEM((TPT,), jnp.int32),
            pltpu.VMEM((TPT,), jnp.float32),
            pltpu.VMEM((TPT,), jnp.float32),
            pltpu.VMEM((TPT,), jnp.int32),
            pltpu.VMEM((TPT,), jnp.int32),
            pltpu.VMEM((TPT,), jnp.int32),
            pltpu.VMEM((16,), jnp.int32),
            pltpu.VMEM((NTILES * 16,), jnp.int32),
            pltpu.VMEM((16,), jnp.int32),
            pltpu.VMEM((16,), jnp.int32),
            pltpu.VMEM((ZCH,), jnp.int32),
            pltpu.VMEM((ZCH,), jnp.float32),
            pltpu.SemaphoreType.DMA,
            pltpu.SemaphoreType.DMA,
            pltpu.SemaphoreType.DMA,
            pltpu.SemaphoreType.DMA,
            pltpu.SemaphoreType.DMA,
            pltpu.SemaphoreType.DMA,
        ],
        compiler_params=pltpu.CompilerParams(needs_layout_passes=False),
    )
    return f(gt)[:4]


def _gather_body(hs_hbm, tok_hbm, xs_hbm, idx_v, rows_a, rows_b,
                 sem_a, sem_b, sem_wa, sem_wb):
    wid = lax.axis_index("s") * 2 + lax.axis_index("c")
    base = wid * RPW
    pltpu.sync_copy(tok_hbm.at[pl.ds(base, RPW)], idx_v)
    g0 = pltpu.async_copy(hs_hbm.at[idx_v.at[pl.ds(0, GCH)]], rows_a, sem_a)
    g1 = pltpu.async_copy(hs_hbm.at[idx_v.at[pl.ds(GCH, GCH)]], rows_b, sem_b)
    g0.wait()
    w0 = pltpu.async_copy(rows_a, xs_hbm.at[pl.ds(base, GCH)], sem_wa)
    g1.wait()
    w1 = pltpu.async_copy(rows_b, xs_hbm.at[pl.ds(base + GCH, GCH)], sem_wb)
    w0.wait()
    g2 = pltpu.async_copy(hs_hbm.at[idx_v.at[pl.ds(2 * GCH, GCH)]], rows_a, sem_a)
    g2.wait()
    w2 = pltpu.async_copy(rows_a, xs_hbm.at[pl.ds(base + 2 * GCH, GCH)], sem_wa)
    w1.wait()
    w2.wait()


def _gather(hs, tok_sorted):
    mesh = plsc.VectorSubcoreMesh(core_axis_name="c", subcore_axis_name="s")
    f = pl.kernel(
        _gather_body,
        out_type=jax.ShapeDtypeStruct((ROWS, H), jnp.float32),
        mesh=mesh,
        scratch_types=[
            pltpu.VMEM((RPW,), jnp.int32),
            pltpu.VMEM((GCH, H), jnp.float32),
            pltpu.VMEM((GCH, H), jnp.float32),
            pltpu.SemaphoreType.DMA,
            pltpu.SemaphoreType.DMA,
            pltpu.SemaphoreType.DMA,
            pltpu.SemaphoreType.DMA,
        ],
        compiler_params=pltpu.CompilerParams(needs_layout_passes=False),
    )
    return f(hs, tok_sorted)


def _gmm_body(incl_ref, x_ref, w1_ref, w2_ref, ws_ref, o_ref, w1b_v, w2b_v):
    b = pl.program_id(0)
    tb = incl_ref[E - 1] >> LOG2_BLOCK

    @pl.when(b < tb)
    def _():
        eb = _exp_of(b, incl_ref)
        ebp = _exp_of(b - 1, incl_ref)

        @pl.when((b == 0) | (eb != ebp))
        def _():
            w1b_v[...] = w1_ref[0].astype(jnp.bfloat16)
            w2b_v[...] = w2_ref[0].astype(jnp.bfloat16)

        xb = x_ref[...].astype(jnp.bfloat16)
        h = lax.dot_general(xb, w1b_v[...], (((1,), (1,)), ((), ())),
                            preferred_element_type=jnp.float32)
        x1 = h[:, :I]
        x2 = h[:, I:]
        act = (x1 * lax.logistic(x1) * x2).astype(jnp.bfloat16)
        o = lax.dot_general(act, w2b_v[...], (((1,), (1,)), ((), ())),
                            preferred_element_type=jnp.float32)
        o_ref[...] = o * ws_ref[...]


def _clamp_blk(b, r):
    tb = r[E - 1] >> LOG2_BLOCK
    return jnp.where(b < tb, b, tb - 1)


def _exp_of(b, r):
    bc = _clamp_blk(b, r) * BLOCK
    e = jnp.int32(0)
    for i in range(E):
        e = e + (bc >= r[i]).astype(jnp.int32)
    return e


def _gmm(x_sorted, w1, w2, w_sorted, incl16):
    grid_spec = pltpu.PrefetchScalarGridSpec(
        num_scalar_prefetch=1,
        grid=(NBLK,),
        in_specs=[
            pl.BlockSpec((BLOCK, H), lambda b, r: (_clamp_blk(b, r), 0)),
            pl.BlockSpec((1, 2 * I, H), lambda b, r: (_exp_of(b, r), 0, 0)),
            pl.BlockSpec((1, H, I), lambda b, r: (_exp_of(b, r), 0, 0)),
            pl.BlockSpec((BLOCK, 1), lambda b, r: (_clamp_blk(b, r), 0)),
        ],
        out_specs=pl.BlockSpec((BLOCK, H), lambda b, r: (b, 0)),
        scratch_shapes=[
            pltpu.VMEM((2 * I, H), jnp.bfloat16),
            pltpu.VMEM((H, I), jnp.bfloat16),
        ],
    )
    return pl.pallas_call(
        _gmm_body,
        grid_spec=grid_spec,
        out_shape=jax.ShapeDtypeStruct((ROWS, H), jnp.float32),
    )(incl16, x_sorted, w1, w2, w_sorted.reshape(ROWS, 1))


def _combine_body(os_hbm, pos_hbm, out_hbm, idx_v, a_v, b_v, sem):
    wid = lax.axis_index("s") * 2 + lax.axis_index("c")
    t0 = wid * TPW
    pltpu.sync_copy(pos_hbm.at[0, pl.ds(t0, TPW)], idx_v)
    pltpu.async_copy(os_hbm.at[idx_v], a_v, sem).wait()
    pltpu.sync_copy(pos_hbm.at[1, pl.ds(t0, TPW)], idx_v)
    pltpu.async_copy(os_hbm.at[idx_v], b_v, sem).wait()

    def body(i, carry):
        for ch in range(H // 16):
            sl = pl.ds(ch * 16, 16)
            a_v[i, sl] = a_v[i, sl] + b_v[i, sl]
        return carry

    lax.fori_loop(0, TPW, body, 0)
    pltpu.sync_copy(a_v, out_hbm.at[pl.ds(t0, TPW)])


def _combine(out_sorted, pos):
    mesh = plsc.VectorSubcoreMesh(core_axis_name="c", subcore_axis_name="s")
    f = pl.kernel(
        _combine_body,
        out_type=jax.ShapeDtypeStruct((T, H), jnp.float32),
        mesh=mesh,
        scratch_types=[
            pltpu.VMEM((TPW,), jnp.int32),
            pltpu.VMEM((TPW, H), jnp.float32),
            pltpu.VMEM((TPW, H), jnp.float32),
            pltpu.SemaphoreType.DMA,
        ],
        compiler_params=pltpu.CompilerParams(needs_layout_passes=False),
    )
    return f(out_sorted, pos)


def kernel(hidden_states, w1, w2, gating_output):
    orig_shape = hidden_states.shape
    hs = hidden_states.reshape(T, H)
    gt = gating_output.reshape(T, E).T
    tok_sorted, w_sorted, pos, incl16 = _routing(gt)
    x_sorted = _gather(hs, tok_sorted)
    out_sorted = _gmm(x_sorted, w1, w2, w_sorted, incl16)
    out = _combine(out_sorted, pos)
    return out.reshape(orig_shape)


# speedup vs baseline: 1.6402x; 1.0074x over previous
import jax
import jax.numpy as jnp
from jax import lax
from jax.experimental import pallas as pl
from jax.experimental.pallas import tpu as pltpu
from jax.experimental.pallas import tpu_sc as plsc

T = 2048
E = 8
H = 768
I = 1024
K = 2
P = T * K
BLOCK = 256
NBLK = P // BLOCK + E
ROWS = NBLK * BLOCK
LOG2_BLOCK = 8

NTILES = 16
TPT = T // NTILES
ZCH = ROWS // NTILES
NW = 32
RPW = ROWS // NW
GCH = 64
TPW = T // NW

def _splat(ref, e):
    return plsc.load_gather(ref, [jnp.full((16,), e, jnp.int32)])


def _routing_body(gt_hbm, tok_hbm, wsort_hbm, pos_hbm, incl_hbm, cnts_hbm,
                  g_v, e0_v, e1_v, w0_v, w1_v, p0_v, p1_v, tok_v,
                  cnt_v, allcnt_v, base_v, incl_v, zi_v, zf_v,
                  sem, sem2, sem3, sem4, sem5, sem6):
    c = lax.axis_index("c")

    @pl.when(c == 0)
    def _():
        w = lax.axis_index("s")
        base = w * TPT
        lane = lax.iota(jnp.int32, 16)
        wv = jnp.broadcast_to(w, (16,)).astype(jnp.int32)
        ones16 = jnp.ones((16,), jnp.int32)

        pltpu.sync_copy(gt_hbm, g_v)
        cnt_v[...] = jnp.zeros((16,), jnp.int32)

        def pass_a(j, carry):
            g = [g_v[e, pl.ds(base + j * 16, 16)] for e in range(E)]
            m1 = g[0]
            i1 = jnp.zeros((16,), jnp.int32)
            for e in range(1, E):
                gt = g[e] > m1
                m1 = jnp.where(gt, g[e], m1)
                i1 = jnp.where(gt, e, i1)
            m2 = jnp.full((16,), -jnp.inf, jnp.float32)
            i2 = jnp.zeros((16,), jnp.int32)
            for e in range(E):
                gt = (i1 != e) & (g[e] > m2)
                m2 = jnp.where(gt, g[e], m2)
                i2 = jnp.where(gt, e, i2)
            wt1 = 1.0 / (1.0 + jnp.exp(m2 - m1))
            sl = pl.ds(j * 16, 16)
            e0_v[sl] = i1
            e1_v[sl] = i2
            w0_v[sl] = wt1
            w1_v[sl] = 1.0 - wt1
            tok_v[sl] = base + j * 16 + lane
            plsc.addupdate_scatter(cnt_v, [i1], ones16)
            plsc.addupdate_scatter(cnt_v, [i2], ones16)
            return carry

        lax.fori_loop(0, TPT // 16, pass_a, 0)

        cpub = pltpu.async_copy(cnt_v, cnts_hbm.at[pl.ds(w * 16, 16)], sem5)

        def zinit(j, carry):
            sl = pl.ds(j * 16, 16)
            zi_v[sl] = (w * ZCH + j * 16 + lane) & (T - 1)
            zf_v[sl] = jnp.zeros((16,), jnp.float32)
            return carry

        lax.fori_loop(0, ZCH // 16, zinit, 0)
        z1 = pltpu.async_copy(zi_v, tok_hbm.at[pl.ds(w * ZCH, ZCH)], sem)
        z2 = pltpu.async_copy(zf_v, wsort_hbm.at[pl.ds(w * ZCH, ZCH)], sem2)
        z1.wait()
        z2.wait()
        cpub.wait()

    plsc.subcore_barrier()

    @pl.when(c == 0)
    def _():
        w = lax.axis_index("s")
        base = w * TPT
        lane = lax.iota(jnp.int32, 16)
        wv = jnp.broadcast_to(w, (16,)).astype(jnp.int32)
        ones16 = jnp.ones((16,), jnp.int32)

        pltpu.sync_copy(cnts_hbm, allcnt_v)
        totals = jnp.zeros((16,), jnp.int32)
        myprefix = jnp.zeros((16,), jnp.int32)
        for r in range(NTILES):
            row = allcnt_v[pl.ds(r * 16, 16)]
            totals = totals + row
            rv = jnp.full((16,), r, jnp.int32)
            myprefix = myprefix + jnp.where(rv < wv, row, 0)
        padded = ((totals + (BLOCK - 1)) >> LOG2_BLOCK) << LOG2_BLOCK
        incl = plsc.cumsum(padded)
        incl_v[...] = incl
        base_v[...] = (incl - padded) + myprefix

        def mk_pass_b(ev_ref, pv_ref):
            def pass_b(j, carry):
                sl = pl.ds(j * 16, 16)
                ev = ev_ref[sl]
                rank = jnp.zeros((16,), jnp.int32)
                for e in range(E):
                    m = ev == e
                    cs = plsc.cumsum(m.astype(jnp.int32))
                    rank = jnp.where(m, cs - 1, rank)
                pv_ref[sl] = plsc.load_gather(base_v, [ev]) + rank
                plsc.addupdate_scatter(base_v, [ev], ones16)
                return carry
            return pass_b

        lax.fori_loop(0, TPT // 16, mk_pass_b(e0_v, p0_v), 0)
        lax.fori_loop(0, TPT // 16, mk_pass_b(e1_v, p1_v), 0)

        c1 = pltpu.async_copy(tok_v, tok_hbm.at[p0_v], sem)
        c2 = pltpu.async_copy(tok_v, tok_hbm.at[p1_v], sem2)
        c3 = pltpu.async_copy(w0_v, wsort_hbm.at[p0_v], sem3)
        c4 = pltpu.async_copy(w1_v, wsort_hbm.at[p1_v], sem4)
        c5 = pltpu.async_copy(p0_v, pos_hbm.at[0, pl.ds(base, TPT)], sem5)
        c6 = pltpu.async_copy(p1_v, pos_hbm.at[1, pl.ds(base, TPT)], sem6)
        c1.wait()
        c2.wait()
        c3.wait()
        c4.wait()
        c5.wait()
        c6.wait()

        @pl.when(w == 0)
        def _():
            pltpu.sync_copy(incl_v, incl_hbm)


def _routing(gt):
    mesh = plsc.VectorSubcoreMesh(core_axis_name="c", subcore_axis_name="s")
    f = pl.kernel(
        _routing_body,
        out_type=(
            jax.ShapeDtypeStruct((ROWS,), jnp.int32),
            jax.ShapeDtypeStruct((ROWS,), jnp.float32),
            jax.ShapeDtypeStruct((K, T), jnp.int32),
            jax.ShapeDtypeStruct((16,), jnp.int32),
            jax.ShapeDtypeStruct((NTILES * 16,), jnp.int32),
        ),
        mesh=mesh,
        scratch_types=[
            pltpu.VMEM((E, T), jnp.float32),
            pltpu.VMEM((TPT,), jnp.int32),
            pltpu.VMEM((TPT,), jnp.int32),
            pltpu.VMEM((TPT,), jnp.float32),
            pltpu.VMEM((TPT,), jnp.float32),
            pltpu.VMEM((TPT,), jnp.int32),
            pltpu.VMEM((TPT,), jnp.int32),
            pltpu.VMEM((TPT,), jnp.int32),
            pltpu.VMEM((16,), jnp.int32),
            pltpu.VMEM((NTILES * 16,), jnp.int32),
            pltpu.VMEM((16,), jnp.int32),
            pltpu.VMEM((16,), jnp.int32),
            pltpu.VMEM((ZCH,), jnp.int32),
            pltpu.VMEM((ZCH,), jnp.float32),
            pltpu.SemaphoreType.DMA,
            pltpu.SemaphoreType.DMA,
            pltpu.SemaphoreType.DMA,
            pltpu.SemaphoreType.DMA,
            pltpu.SemaphoreType.DMA,
            pltpu.SemaphoreType.DMA,
        ],
        compiler_params=pltpu.CompilerParams(needs_layout_passes=False),
    )
    return f(gt)[:4]


def _gather_body(hs_hbm, tok_hbm, xs_hbm, idx_v, rows_a, rows_b,
                 sem_a, sem_b, sem_wa, sem_wb):
    wid = lax.axis_index("s") * 2 + lax.axis_index("c")
    base = wid * RPW
    pltpu.sync_copy(tok_hbm.at[pl.ds(base, RPW)], idx_v)
    g0 = pltpu.async_copy(hs_hbm.at[idx_v.at[pl.ds(0, GCH)]], rows_a, sem_a)
    g1 = pltpu.async_copy(hs_hbm.at[idx_v.at[pl.ds(GCH, GCH)]], rows_b, sem_b)
    g0.wait()
    w0 = pltpu.async_copy(rows_a, xs_hbm.at[pl.ds(base, GCH)], sem_wa)
    g1.wait()
    w1 = pltpu.async_copy(rows_b, xs_hbm.at[pl.ds(base + GCH, GCH)], sem_wb)
    w0.wait()
    g2 = pltpu.async_copy(hs_hbm.at[idx_v.at[pl.ds(2 * GCH, GCH)]], rows_a, sem_a)
    g2.wait()
    w2 = pltpu.async_copy(rows_a, xs_hbm.at[pl.ds(base + 2 * GCH, GCH)], sem_wa)
    w1.wait()
    w2.wait()


def _gather(hs, tok_sorted):
    mesh = plsc.VectorSubcoreMesh(core_axis_name="c", subcore_axis_name="s")
    f = pl.kernel(
        _gather_body,
        out_type=jax.ShapeDtypeStruct((ROWS, H), jnp.float32),
        mesh=mesh,
        scratch_types=[
            pltpu.VMEM((RPW,), jnp.int32),
            pltpu.VMEM((GCH, H), jnp.float32),
            pltpu.VMEM((GCH, H), jnp.float32),
            pltpu.SemaphoreType.DMA,
            pltpu.SemaphoreType.DMA,
            pltpu.SemaphoreType.DMA,
            pltpu.SemaphoreType.DMA,
        ],
        compiler_params=pltpu.CompilerParams(needs_layout_passes=False),
    )
    return f(hs, tok_sorted)


def _gmm_body(incl_ref, x_ref, w1_ref, w2_ref, ws_ref, o_ref, w1b_v, w2b_v):
    b = pl.program_id(0)
    tb = incl_ref[E - 1] >> LOG2_BLOCK

    @pl.when(b < tb)
    def _():
        eb = _exp_of(b, incl_ref)
        ebp = _exp_of(b - 1, incl_ref)

        @pl.when((b == 0) | (eb != ebp))
        def _():
            w1b_v[...] = w1_ref[0].astype(jnp.bfloat16)
            w2b_v[...] = w2_ref[0].astype(jnp.bfloat16)

        xb = x_ref[...].astype(jnp.bfloat16)
        h = lax.dot_general(xb, w1b_v[...], (((1,), (1,)), ((), ())),
                            preferred_element_type=jnp.float32)
        x1 = h[:, :I]
        x2 = h[:, I:]
        act = (x1 * lax.logistic(x1) * x2).astype(jnp.bfloat16)
        o = lax.dot_general(act, w2b_v[...], (((1,), (1,)), ((), ())),
                            preferred_element_type=jnp.float32)
        o_ref[...] = o * ws_ref[...]


def _clamp_blk(b, r):
    tb = r[E - 1] >> LOG2_BLOCK
    return jnp.where(b < tb, b, tb - 1)


def _exp_of(b, r):
    bc = _clamp_blk(b, r) * BLOCK
    e = jnp.int32(0)
    for i in range(E):
        e = e + (bc >= r[i]).astype(jnp.int32)
    return e


def _gmm(x_sorted, w1, w2, w_sorted, incl16):
    grid_spec = pltpu.PrefetchScalarGridSpec(
        num_scalar_prefetch=1,
        grid=(NBLK,),
        in_specs=[
            pl.BlockSpec((BLOCK, H), lambda b, r: (_clamp_blk(b, r), 0)),
            pl.BlockSpec((1, 2 * I, H), lambda b, r: (_exp_of(b, r), 0, 0)),
            pl.BlockSpec((1, H, I), lambda b, r: (_exp_of(b, r), 0, 0)),
            pl.BlockSpec((BLOCK, 1), lambda b, r: (_clamp_blk(b, r), 0)),
        ],
        out_specs=pl.BlockSpec((BLOCK, H), lambda b, r: (b, 0)),
        scratch_shapes=[
            pltpu.VMEM((2 * I, H), jnp.bfloat16),
            pltpu.VMEM((H, I), jnp.bfloat16),
        ],
    )
    return pl.pallas_call(
        _gmm_body,
        grid_spec=grid_spec,
        out_shape=jax.ShapeDtypeStruct((ROWS, H), jnp.float32),
    )(incl16, x_sorted, w1, w2, w_sorted.reshape(ROWS, 1))


def _combine_body(os_hbm, pos_hbm, out_hbm, idx_v, a_v, b_v, sem):
    wid = lax.axis_index("s") * 2 + lax.axis_index("c")
    t0 = wid * TPW
    pltpu.sync_copy(pos_hbm.at[0, pl.ds(t0, TPW)], idx_v)
    pltpu.async_copy(os_hbm.at[idx_v], a_v, sem).wait()
    pltpu.sync_copy(pos_hbm.at[1, pl.ds(t0, TPW)], idx_v)
    pltpu.async_copy(os_hbm.at[idx_v], b_v, sem).wait()

    def body(i, carry):
        for ch in range(H // 16):
            sl = pl.ds(ch * 16, 16)
            a_v[i, sl] = a_v[i, sl] + b_v[i, sl]
        return carry

    lax.fori_loop(0, TPW, body, 0)
    pltpu.sync_copy(a_v, out_hbm.at[pl.ds(t0, TPW)])


def _combine(out_sorted, pos):
    mesh = plsc.VectorSubcoreMesh(core_axis_name="c", subcore_axis_name="s")
    f = pl.kernel(
        _combine_body,
        out_type=jax.ShapeDtypeStruct((T, H), jnp.float32),
        mesh=mesh,
        scratch_types=[
            pltpu.VMEM((TPW,), jnp.int32),
            pltpu.VMEM((TPW, H), jnp.float32),
            pltpu.VMEM((TPW, H), jnp.float32),
            pltpu.SemaphoreType.DMA,
        ],
        compiler_params=pltpu.CompilerParams(needs_layout_passes=False),
    )
    return f(out_sorted, pos)


def kernel(hidden_states, w1, w2, gating_output):
    orig_shape = hidden_states.shape
    hs = hidden_states.reshape(T, H)
    gt = gating_output.reshape(T, E).T
    tok_sorted, w_sorted, pos, incl16 = _routing(gt)
    x_sorted = _gather(hs, tok_sorted)
    out_sorted = _gmm(x_sorted, w1, w2, w_sorted, incl16)
    out = _combine(out_sorted, pos)
    return out.reshape(orig_shape)
